# Initial kernel scaffold; baseline (speedup 1.0000x reference)
import jax
import jax.numpy as jnp
from jax.experimental import pallas as pl


def kernel(x, spiral0, spiral1, spiral2, down_idx0, down_w0, down_idx1, down_w1, down_idx2, down_w2, W0, b0, W1, b1, W2, b2, W3, b3, W4, b4):
    raise NotImplementedError("write your pallas kernel here")


# speedup vs baseline: 4.2739x; 4.2739x over previous
import jax
import jax.numpy as jnp
from jax import lax
from jax.experimental import pallas as pl
from jax.experimental.pallas import tpu as pltpu
from jax.experimental.pallas import tpu_sc as plsc

NC, NS = 2, 16
NW = NC * NS

_B = 8
_N0, _N1, _N2, _N3 = 50000, 12500, 3125, 784
_L = 9
_K = 3


def _rup(n, m):
    return ((n + m - 1) // m) * m


def _elu(z):
    return jnp.where(z > 0, z, jnp.exp(jnp.minimum(z, 0.0)) - 1.0)


def _sc_mesh():
    return plsc.VectorSubcoreMesh(
        core_axis_name="c", subcore_axis_name="s", num_cores=NC, num_subcores=NS
    )


def _make_gather(N, D, L, Np, c):
    ncs = Np // c
    total = L * ncs
    nj = -(-total // NW)

    def body(table, idx, out, idx_v, rows_v, sem):
        wid = lax.axis_index("s") * NC + lax.axis_index("c")

        def step(t, carry):
            j = t * NW + wid

            @pl.when(j < total)
            def _do():
                l = j // ncs
                o = (j % ncs) * c
                pltpu.sync_copy(idx.at[l, pl.ds(o, c)], idx_v)
                pltpu.async_copy(table.at[idx_v], rows_v, sem).wait()
                pltpu.sync_copy(rows_v, out.at[l, pl.ds(o, c), :])

            return carry

        lax.fori_loop(0, nj, step, 0)

    return pl.kernel(
        body,
        out_type=jax.ShapeDtypeStruct((L, Np, D), jnp.float32),
        mesh=_sc_mesh(),
        scratch_types=[
            pltpu.VMEM((c,), jnp.int32),
            pltpu.VMEM((c, D), jnp.float32),
            pltpu.SemaphoreType.DMA,
        ],
        compiler_params=pltpu.CompilerParams(use_tc_tiling_on_sc=False),
    )


def _make_pool(N, D, Mp, c):
    ncs = Mp // c
    nj = -(-ncs // NW)

    def body(table, idx, wts, out, idx_v, a0, a1, a2, wb0, wb1, wb2, sem):
        wid = lax.axis_index("s") * NC + lax.axis_index("c")
        bufs = (a0, a1, a2)
        wbufs = (wb0, wb1, wb2)

        def step(t, carry):
            j = t * NW + wid

            @pl.when(j < ncs)
            def _do():
                o = j * c
                for k in range(_K):
                    pltpu.sync_copy(idx.at[k, pl.ds(o, c)], idx_v)
                    pltpu.async_copy(table.at[idx_v], bufs[k], sem).wait()
                    pltpu.sync_copy(wts.at[k, pl.ds(o, c), :], wbufs[k])

                def row(r, carry2):
                    w0 = wb0[r, :]
                    w1 = wb1[r, :]
                    w2 = wb2[r, :]
                    for q in range(D // 16):
                        s = q * 16
                        v = (
                            a0[r, pl.ds(s, 16)] * w0
                            + a1[r, pl.ds(s, 16)] * w1
                            + a2[r, pl.ds(s, 16)] * w2
                        )
                        a0[r, pl.ds(s, 16)] = v
                    return carry2

                lax.fori_loop(0, c, row, 0)
                pltpu.sync_copy(a0, out.at[pl.ds(o, c), :])

            return carry

        lax.fori_loop(0, nj, step, 0)

    return pl.kernel(
        body,
        out_type=jax.ShapeDtypeStruct((Mp, D), jnp.float32),
        mesh=_sc_mesh(),
        scratch_types=[
            pltpu.VMEM((c,), jnp.int32),
            pltpu.VMEM((c, D), jnp.float32),
            pltpu.VMEM((c, D), jnp.float32),
            pltpu.VMEM((c, D), jnp.float32),
            pltpu.VMEM((c, 16), jnp.float32),
            pltpu.VMEM((c, 16), jnp.float32),
            pltpu.VMEM((c, 16), jnp.float32),
            pltpu.SemaphoreType.DMA,
        ],
        compiler_params=pltpu.CompilerParams(use_tc_tiling_on_sc=False),
    )


def _make_matmul(NR, NRp, Cin, Cout, VB):
    grid = -(-NR // VB)

    def body(*refs):
        g_refs = refs[:_L]
        w_ref, b_ref, out_ref = refs[_L], refs[_L + 1], refs[_L + 2]
        cat = jnp.concatenate([r[0] for r in g_refs], axis=1)
        z = jnp.dot(cat, w_ref[...], preferred_element_type=jnp.float32)
        z = z + b_ref[...]
        out_ref[...] = _elu(z)

    in_specs = [
        pl.BlockSpec((1, VB, Cin), (lambda i, l=l: (l, i, 0)))
        for l in range(_L)
    ]
    in_specs.append(pl.BlockSpec((_L * Cin, Cout), lambda i: (0, 0)))
    in_specs.append(pl.BlockSpec((1, Cout), lambda i: (0, 0)))

    return pl.pallas_call(
        body,
        grid=(grid,),
        in_specs=in_specs,
        out_specs=pl.BlockSpec((VB, Cout), lambda i: (i, 0)),
        out_shape=jax.ShapeDtypeStruct((NR, Cout), jnp.float32),
        compiler_params=pltpu.CompilerParams(
            dimension_semantics=("arbitrary",)
        ),
    )


def _make_final(Mp):
    steps = 14
    vb = _N3 // steps
    kb = vb * 128

    def body(h_ref, w3_ref, b3_ref, w4_ref, b4_ref, xe_ref, y_ref, acc_ref):
        i = pl.program_id(0)

        @pl.when(i == 0)
        def _init():
            acc_ref[...] = jnp.zeros((_B, 256), jnp.float32)

        def vstep(v, acc):
            hv = h_ref[pl.ds(v * _B, _B), :]
            wv = w3_ref[pl.ds(v * 128, 128), :]
            return acc + jnp.dot(hv, wv, preferred_element_type=jnp.float32)

        acc_ref[...] = lax.fori_loop(0, vb, vstep, acc_ref[...])

        @pl.when(i == steps - 1)
        def _fin():
            xe = _elu(acc_ref[...] + b3_ref[...])
            xe_ref[...] = xe
            y_ref[...] = (
                jnp.dot(xe, w4_ref[...], preferred_element_type=jnp.float32)
                + b4_ref[...]
            )

    return pl.pallas_call(
        body,
        grid=(steps,),
        in_specs=[
            pl.BlockSpec((vb * _B, 128), lambda i: (i, 0)),
            pl.BlockSpec((kb, 256), lambda i: (i, 0)),
            pl.BlockSpec((1, 256), lambda i: (0, 0)),
            pl.BlockSpec((256, 64), lambda i: (0, 0)),
            pl.BlockSpec((1, 64), lambda i: (0, 0)),
        ],
        out_specs=[
            pl.BlockSpec((_B, 256), lambda i: (0, 0)),
            pl.BlockSpec((_B, 64), lambda i: (0, 0)),
        ],
        out_shape=[
            jax.ShapeDtypeStruct((_B, 256), jnp.float32),
            jax.ShapeDtypeStruct((_B, 64), jnp.float32),
        ],
        scratch_shapes=[pltpu.VMEM((_B, 256), jnp.float32)],
        compiler_params=pltpu.CompilerParams(
            dimension_semantics=("arbitrary",)
        ),
    )


def _pad_idx(idx, Np):
    t = idx.T.astype(jnp.int32)
    return jnp.pad(t, ((0, 0), (0, Np - t.shape[1])))


def _pad_w(w, Np):
    t = w.T.astype(jnp.float32)
    t = jnp.pad(t, ((0, 0), (0, Np - t.shape[1])))
    return jnp.broadcast_to(t[:, :, None], (t.shape[0], Np, 16))


_NP0 = _rup(_N0, 128)
_MP0 = _rup(_N1, 64)
_NP1 = _rup(_N1, 128)
_MP1 = _rup(_N2, 64)
_NP2 = _rup(_N2, 128)
_MP2 = _rup(_N3, 32)


def kernel(x, spiral0, spiral1, spiral2, down_idx0, down_w0, down_idx1,
           down_w1, down_idx2, down_w2, W0, b0, W1, b1, W2, b2, W3, b3,
           W4, b4):
    xt = jnp.pad(x, ((0, 0), (0, 0), (0, 1)))
    xt = xt.transpose(1, 0, 2).reshape(_N0, _B * 4)

    sp0 = _pad_idx(spiral0, _NP0)
    sp1 = _pad_idx(spiral1, _NP1)
    sp2 = _pad_idx(spiral2, _NP2)
    d0 = _pad_idx(down_idx0, _MP0)
    d1 = _pad_idx(down_idx1, _MP1)
    d2 = _pad_idx(down_idx2, _MP2)
    w0p = _pad_w(down_w0, _MP0)
    w1p = _pad_w(down_w1, _MP1)
    w2p = _pad_w(down_w2, _MP2)

    W0p = jnp.pad(W0.reshape(_L, 3, 32), ((0, 0), (0, 1), (0, 0)))
    eyeB = jnp.eye(_B, dtype=jnp.float32)
    W0k = jax.vmap(lambda w: jnp.kron(eyeB, w))(W0p)
    W0k = W0k.reshape(_L * _B * 4, _B * 32)
    b0k = jnp.tile(b0, _B).reshape(1, _B * 32)
    b1r = b1.reshape(1, 64)
    b2r = b2.reshape(1, 128)
    b3r = b3.reshape(1, 256)
    b4r = b4.reshape(1, 64)

    g0 = _make_gather(_N0, _B * 4, _L, _NP0, 128)(xt, sp0)
    h0 = _make_matmul(_N0, _NP0, _B * 4, _B * 32, 1000)(
        *([g0] * _L), W0k, b0k)
    h0p = _make_pool(_N0, _B * 32, _MP0, 64)(h0, d0, w0p)

    g1 = _make_gather(_MP0, _B * 32, _L, _NP1, 128)(h0p, sp1)
    g1r = g1.reshape(_L, _NP1 * _B, 32)
    h1 = _make_matmul(_N1 * _B, _NP1 * _B, 32, 64, 1024)(
        *([g1r] * _L), W1, b1r)
    h1p = _make_pool(_N1, _B * 64, _MP1, 64)(
        h1.reshape(_N1, _B * 64), d1, w1p)

    g2 = _make_gather(_MP1, _B * 64, _L, _NP2, 128)(h1p, sp2)
    g2r = g2.reshape(_L, _NP2 * _B, 64)
    h2 = _make_matmul(_N2 * _B, _NP2 * _B, 64, 128, 1024)(
        *([g2r] * _L), W2, b2r)
    h2p = _make_pool(_N2, _B * 128, _MP2, 32)(
        h2.reshape(_N2, _B * 128), d2, w2p)

    xe, y = _make_final(_MP2)(h2p.reshape(_MP2 * _B, 128), W3, b3r, W4, b4r)
    return (xe, y)


# speedup vs baseline: 7.1125x; 1.6642x over previous
import jax
import jax.numpy as jnp
from jax import lax
from jax.experimental import pallas as pl
from jax.experimental.pallas import tpu as pltpu
from jax.experimental.pallas import tpu_sc as plsc

NC, NS = 2, 16
NW = NC * NS

_B = 8
_N0, _N1, _N2, _N3 = 50000, 12500, 3125, 784
_L = 9
_K = 3


def _rup(n, m):
    return ((n + m - 1) // m) * m


def _elu(z):
    return jnp.where(z > 0, z, jnp.exp(jnp.minimum(z, 0.0)) - 1.0)


def _sc_mesh():
    return plsc.VectorSubcoreMesh(
        core_axis_name="c", subcore_axis_name="s", num_cores=NC, num_subcores=NS
    )


def _make_gather(N, D, L, Np, c):
    ncs = Np // c
    total = L * ncs
    nj = -(-total // NW)

    def body(table, idx, out, idx_v, rows_v, sem):
        wid = lax.axis_index("s") * NC + lax.axis_index("c")

        def step(t, carry):
            j = t * NW + wid

            @pl.when(j < total)
            def _do():
                l = j // ncs
                o = (j % ncs) * c
                pltpu.sync_copy(idx.at[l, pl.ds(o, c)], idx_v)
                pltpu.async_copy(table.at[idx_v], rows_v, sem).wait()
                pltpu.sync_copy(rows_v, out.at[l, pl.ds(o, c), :])

            return carry

        lax.fori_loop(0, nj, step, 0)

    return pl.kernel(
        body,
        out_type=jax.ShapeDtypeStruct((L, Np, D), jnp.float32),
        mesh=_sc_mesh(),
        scratch_types=[
            pltpu.VMEM((c,), jnp.int32),
            pltpu.VMEM((c, D), jnp.float32),
            pltpu.SemaphoreType.DMA,
        ],
        compiler_params=pltpu.CompilerParams(use_tc_tiling_on_sc=False),
    )


def _make_pool(N, D, Mp, c):
    ncs = Mp // c
    nj = -(-ncs // NW)

    def body(table, idx, wts, out, idx_v, a0, a1, a2, wb0, wb1, wb2, sem):
        wid = lax.axis_index("s") * NC + lax.axis_index("c")
        bufs = (a0, a1, a2)
        wbufs = (wb0, wb1, wb2)

        def step(t, carry):
            j = t * NW + wid

            @pl.when(j < ncs)
            def _do():
                o = j * c
                for k in range(_K):
                    pltpu.sync_copy(idx.at[k, pl.ds(o, c)], idx_v)
                    pltpu.async_copy(table.at[idx_v], bufs[k], sem).wait()
                    pltpu.sync_copy(
                        wts.at[pl.ds((k * Mp + o) * 16, c * 16)], wbufs[k])

                def row(r, carry2):
                    w0 = wb0[pl.ds(r * 16, 16)]
                    w1 = wb1[pl.ds(r * 16, 16)]
                    w2 = wb2[pl.ds(r * 16, 16)]
                    for q in range(D // 16):
                        s = q * 16
                        v = (
                            a0[r, pl.ds(s, 16)] * w0
                            + a1[r, pl.ds(s, 16)] * w1
                            + a2[r, pl.ds(s, 16)] * w2
                        )
                        a0[r, pl.ds(s, 16)] = v
                    return carry2

                lax.fori_loop(0, c, row, 0)
                pltpu.sync_copy(a0, out.at[pl.ds(o, c), :])

            return carry

        lax.fori_loop(0, nj, step, 0)

    return pl.kernel(
        body,
        out_type=jax.ShapeDtypeStruct((Mp, D), jnp.float32),
        mesh=_sc_mesh(),
        scratch_types=[
            pltpu.VMEM((c,), jnp.int32),
            pltpu.VMEM((c, D), jnp.float32),
            pltpu.VMEM((c, D), jnp.float32),
            pltpu.VMEM((c, D), jnp.float32),
            pltpu.VMEM((c * 16,), jnp.float32),
            pltpu.VMEM((c * 16,), jnp.float32),
            pltpu.VMEM((c * 16,), jnp.float32),
            pltpu.SemaphoreType.DMA,
        ],
        compiler_params=pltpu.CompilerParams(use_tc_tiling_on_sc=False),
    )


def _make_m0(VB):
    nrows = _NP0 * 32 // 128
    nv4 = _N0 // 4
    grid = -(-nv4 // VB)

    def body(*refs):
        g_refs = refs[:_L]
        w_ref, b_ref, out_ref = refs[_L], refs[_L + 1], refs[_L + 2]
        vals = [r[0] for r in g_refs]
        res = []
        for dv in range(4):
            cat = jnp.concatenate(
                [v[:, dv * 32:(dv + 1) * 32] for v in vals], axis=1)
            z = jnp.dot(cat, w_ref[...], preferred_element_type=jnp.float32)
            res.append(_elu(z + b_ref[...]))
        rows = [res[r // 2][:, None, (r % 2) * 128:(r % 2 + 1) * 128]
                for r in range(8)]
        out = jnp.concatenate(rows, axis=1)
        out_ref[...] = out.reshape(VB * 8, 128)

    in_specs = [
        pl.BlockSpec((1, VB, 128), (lambda i, l=l: (l, i, 0)))
        for l in range(_L)
    ]
    in_specs.append(pl.BlockSpec((_L * 32, 256), lambda i: (0, 0)))
    in_specs.append(pl.BlockSpec((1, 256), lambda i: (0, 0)))

    return pl.pallas_call(
        body,
        grid=(grid,),
        in_specs=in_specs,
        out_specs=pl.BlockSpec((VB * 8, 128), lambda i: (i, 0)),
        out_shape=jax.ShapeDtypeStruct((nv4 * 8, 128), jnp.float32),
        compiler_params=pltpu.CompilerParams(
            dimension_semantics=("arbitrary",)
        ),
    )


def _make_mm(N, Np, Cin, Cout, VBv):
    rv_in = 8 * Cin // 128
    per_row = 128 // Cin
    zc = per_row * Cout
    spr = zc // 128
    rv_out = rv_in * spr
    grid = -(-N // VBv)

    def body(*refs):
        g_refs = refs[:_L]
        w_ref, b_ref, out_ref = refs[_L], refs[_L + 1], refs[_L + 2]
        vals = [r[0].reshape(VBv, rv_in, 128) for r in g_refs]
        rows = []
        for h in range(rv_in):
            cat = jnp.concatenate([v[:, h, :] for v in vals], axis=1)
            z = jnp.dot(cat, w_ref[...], preferred_element_type=jnp.float32)
            z = _elu(z + b_ref[...])
            for s in range(spr):
                rows.append(z[:, None, s * 128:(s + 1) * 128])
        out = jnp.concatenate(rows, axis=1)
        out_ref[...] = out.reshape(VBv * rv_out, 128)

    in_specs = [
        pl.BlockSpec((1, VBv * rv_in, 128), (lambda i, l=l: (l, i, 0)))
        for l in range(_L)
    ]
    in_specs.append(pl.BlockSpec((_L * 128, zc), lambda i: (0, 0)))
    in_specs.append(pl.BlockSpec((1, zc), lambda i: (0, 0)))

    return pl.pallas_call(
        body,
        grid=(grid,),
        in_specs=in_specs,
        out_specs=pl.BlockSpec((VBv * rv_out, 128), lambda i: (i, 0)),
        out_shape=jax.ShapeDtypeStruct((N * rv_out, 128), jnp.float32),
        compiler_params=pltpu.CompilerParams(
            dimension_semantics=("arbitrary",)
        ),
    )


def _make_final(Mp):
    steps = 14
    vb = _N3 // steps
    kb = vb * 128

    def body(h_ref, w3_ref, b3_ref, w4_ref, b4_ref, xe_ref, y_ref, acc_ref):
        i = pl.program_id(0)

        @pl.when(i == 0)
        def _init():
            acc_ref[...] = jnp.zeros((_B, 256), jnp.float32)

        def vstep(v, acc):
            hv = h_ref[pl.ds(v * _B, _B), :]
            wv = w3_ref[pl.ds(v * 128, 128), :]
            return acc + jnp.dot(hv, wv, preferred_element_type=jnp.float32)

        acc_ref[...] = lax.fori_loop(0, vb, vstep, acc_ref[...])

        @pl.when(i == steps - 1)
        def _fin():
            xe = _elu(acc_ref[...] + b3_ref[...])
            xe_ref[...] = xe
            y_ref[...] = (
                jnp.dot(xe, w4_ref[...], preferred_element_type=jnp.float32)
                + b4_ref[...]
            )

    return pl.pallas_call(
        body,
        grid=(steps,),
        in_specs=[
            pl.BlockSpec((vb * _B, 128), lambda i: (i, 0)),
            pl.BlockSpec((kb, 256), lambda i: (i, 0)),
            pl.BlockSpec((1, 256), lambda i: (0, 0)),
            pl.BlockSpec((256, 64), lambda i: (0, 0)),
            pl.BlockSpec((1, 64), lambda i: (0, 0)),
        ],
        out_specs=[
            pl.BlockSpec((_B, 256), lambda i: (0, 0)),
            pl.BlockSpec((_B, 64), lambda i: (0, 0)),
        ],
        out_shape=[
            jax.ShapeDtypeStruct((_B, 256), jnp.float32),
            jax.ShapeDtypeStruct((_B, 64), jnp.float32),
        ],
        scratch_shapes=[pltpu.VMEM((_B, 256), jnp.float32)],
        compiler_params=pltpu.CompilerParams(
            dimension_semantics=("arbitrary",)
        ),
    )


def _pad_idx(idx, Np):
    t = idx.T.astype(jnp.int32)
    return jnp.pad(t, ((0, 0), (0, Np - t.shape[1])))


def _pad_w(w, Np):
    t = w.T.astype(jnp.float32)
    t = jnp.pad(t, ((0, 0), (0, Np - t.shape[1])))
    return jnp.broadcast_to(t[:, :, None], (t.shape[0], Np, 16)).reshape(-1)


_NP0 = _rup(_N0, 128)
_MP0 = _rup(_N1, 64)
_NP1 = _rup(_N1, 128)
_MP1 = _rup(_N2, 64)
_NP2 = _rup(_N2, 128)
_MP2 = _rup(_N3, 32)


def kernel(x, spiral0, spiral1, spiral2, down_idx0, down_w0, down_idx1,
           down_w1, down_idx2, down_w2, W0, b0, W1, b1, W2, b2, W3, b3,
           W4, b4):
    xt = jnp.pad(x, ((0, 0), (0, 0), (0, 1)))
    xt = xt.transpose(1, 0, 2).reshape(_N0, _B * 4)

    sp0 = _pad_idx(spiral0, _NP0)
    sp1 = _pad_idx(spiral1, _NP1)
    sp2 = _pad_idx(spiral2, _NP2)
    d0 = _pad_idx(down_idx0, _MP0)
    d1 = _pad_idx(down_idx1, _MP1)
    d2 = _pad_idx(down_idx2, _MP2)
    w0p = _pad_w(down_w0, _MP0)
    w1p = _pad_w(down_w1, _MP1)
    w2p = _pad_w(down_w2, _MP2)

    W0p = jnp.pad(W0.reshape(_L, 3, 32), ((0, 0), (0, 1), (0, 0)))
    eyeB = jnp.eye(_B, dtype=jnp.float32)
    W0k = jax.vmap(lambda w: jnp.kron(eyeB, w))(W0p)
    W0k = W0k.reshape(_L * _B * 4, _B * 32)
    b0k = jnp.tile(b0, _B).reshape(1, _B * 32)
    eye4 = jnp.eye(4, dtype=jnp.float32)
    eye2 = jnp.eye(2, dtype=jnp.float32)
    Wk1 = jax.vmap(lambda w: jnp.kron(eye4, w))(W1.reshape(_L, 32, 64))
    Wk1 = Wk1.reshape(_L * 128, 256)
    bk1 = jnp.tile(b1, 4).reshape(1, 256)
    Wk2 = jax.vmap(lambda w: jnp.kron(eye2, w))(W2.reshape(_L, 64, 128))
    Wk2 = Wk2.reshape(_L * 128, 256)
    bk2 = jnp.tile(b2, 2).reshape(1, 256)
    b3r = b3.reshape(1, 256)
    b4r = b4.reshape(1, 64)

    g0 = _make_gather(_N0, _B * 4, _L, _NP0, 128)(xt, sp0)
    h0 = _make_m0(512)(*([g0.reshape(_L, _NP0 * 32 // 128, 128)] * _L),
                       W0k, b0k)
    h0p = _make_pool(_N0, _B * 32, _MP0, 64)(
        h0.reshape(_N0, _B * 32), d0, w0p)

    g1 = _make_gather(_MP0, _B * 32, _L, _NP1, 128)(h0p, sp1)
    h1 = _make_mm(_N1, _NP1, 32, 64, 512)(
        *([g1.reshape(_L, _NP1 * 2, 128)] * _L), Wk1, bk1)
    h1p = _make_pool(_N1, _B * 64, _MP1, 64)(
        h1.reshape(_N1, _B * 64), d1, w1p)

    g2 = _make_gather(_MP1, _B * 64, _L, _NP2, 128)(h1p, sp2)
    h2 = _make_mm(_N2, _NP2, 64, 128, 256)(
        *([g2.reshape(_L, _NP2 * 4, 128)] * _L), Wk2, bk2)
    h2p = _make_pool(_N2, _B * 128, _MP2, 32)(
        h2.reshape(_N2, _B * 128), d2, w2p)

    xe, y = _make_final(_MP2)(h2p.reshape(_MP2 * _B, 128), W3, b3r, W4, b4r)
    return (xe, y)


# speedup vs baseline: 8.0249x; 1.1283x over previous
import jax
import jax.numpy as jnp
from jax import lax
from jax.experimental import pallas as pl
from jax.experimental.pallas import tpu as pltpu
from jax.experimental.pallas import tpu_sc as plsc

NC, NS = 2, 16
NW = NC * NS

_B = 8
_N0, _N1, _N2, _N3 = 50000, 12500, 3125, 784
_L = 9
_K = 3


def _rup(n, m):
    return ((n + m - 1) // m) * m


def _elu(z):
    return jnp.where(z > 0, z, jnp.exp(jnp.minimum(z, 0.0)) - 1.0)


def _sc_mesh():
    return plsc.VectorSubcoreMesh(
        core_axis_name="c", subcore_axis_name="s", num_cores=NC, num_subcores=NS
    )


def _make_gather(N, D, L, Np, c):
    total = L * Np // c
    nj = -(-total // NW)
    nj += nj % 2
    ncs_pad = nj * NW

    def body(table, idx, out, ib, b0, b1, gs0, gs1, ss0, ss1):
        wid = lax.axis_index("s") * NC + lax.axis_index("c")
        base = wid * nj
        bufs = (b0, b1)
        gsem = (gs0, gs1)
        ssem = (ss0, ss1)
        pltpu.sync_copy(idx.at[pl.ds(base, nj), :], ib)

        def gather(t, b):
            pltpu.async_copy(table.at[ib.at[t]], bufs[b], gsem[b])

        def store(t, b):
            pltpu.async_copy(
                bufs[b], out.at[pl.ds((base + t) * c, c), :], ssem[b])

        @pl.when(base < total)
        def _p0():
            gather(0, 0)

        @pl.when(base + 1 < total)
        def _p1():
            gather(1, 1)

        def grp(g, carry):
            for b in range(2):
                t = g * 2 + b

                @pl.when(base + t < total)
                def _fin(t=t, b=b):
                    pltpu.make_async_copy(
                        table.at[ib.at[t]], bufs[b], gsem[b]).wait()
                    store(t, b)

                @pl.when((base + t + 2 < total) & (t + 2 < nj))
                def _nxt(t=t, b=b):
                    pltpu.make_async_copy(
                        bufs[b], out.at[pl.ds(0, c), :], ssem[b]).wait()
                    gather(t + 2, b)

            return carry

        lax.fori_loop(0, nj // 2, grp, 0)

        for b in range(2):
            @pl.when(base + b < total)
            def _dr(b=b):
                pltpu.make_async_copy(
                    bufs[b], out.at[pl.ds(0, c), :], ssem[b]).wait()

    return pl.kernel(
        body,
        out_type=jax.ShapeDtypeStruct((L * Np, D), jnp.float32),
        mesh=_sc_mesh(),
        scratch_types=[
            pltpu.VMEM((nj, c), jnp.int32),
            pltpu.VMEM((c, D), jnp.float32),
            pltpu.VMEM((c, D), jnp.float32),
            pltpu.SemaphoreType.DMA,
            pltpu.SemaphoreType.DMA,
            pltpu.SemaphoreType.DMA,
            pltpu.SemaphoreType.DMA,
        ],
        compiler_params=pltpu.CompilerParams(use_tc_tiling_on_sc=False),
    ), ncs_pad


def _make_pool(N, D, Mp, c):
    ncs = Mp // c
    nj = -(-ncs // NW)

    def body(table, idx, wts, out, idx_v, a0, a1, a2, wb0, wb1, wb2, sem):
        wid = lax.axis_index("s") * NC + lax.axis_index("c")
        bufs = (a0, a1, a2)
        wbufs = (wb0, wb1, wb2)

        def step(t, carry):
            j = t * NW + wid

            @pl.when(j < ncs)
            def _do():
                o = j * c
                for k in range(_K):
                    pltpu.sync_copy(idx.at[k, pl.ds(o, c)], idx_v)
                    pltpu.async_copy(table.at[idx_v], bufs[k], sem).wait()
                    pltpu.sync_copy(
                        wts.at[pl.ds((k * Mp + o) * 16, c * 16)], wbufs[k])

                def row(r, carry2):
                    w0 = wb0[pl.ds(r * 16, 16)]
                    w1 = wb1[pl.ds(r * 16, 16)]
                    w2 = wb2[pl.ds(r * 16, 16)]
                    for q in range(D // 16):
                        s = q * 16
                        v = (
                            a0[r, pl.ds(s, 16)] * w0
                            + a1[r, pl.ds(s, 16)] * w1
                            + a2[r, pl.ds(s, 16)] * w2
                        )
                        a0[r, pl.ds(s, 16)] = v
                    return carry2

                lax.fori_loop(0, c, row, 0)
                pltpu.sync_copy(a0, out.at[pl.ds(o, c), :])

            return carry

        lax.fori_loop(0, nj, step, 0)

    return pl.kernel(
        body,
        out_type=jax.ShapeDtypeStruct((Mp, D), jnp.float32),
        mesh=_sc_mesh(),
        scratch_types=[
            pltpu.VMEM((c,), jnp.int32),
            pltpu.VMEM((c, D), jnp.float32),
            pltpu.VMEM((c, D), jnp.float32),
            pltpu.VMEM((c, D), jnp.float32),
            pltpu.VMEM((c * 16,), jnp.float32),
            pltpu.VMEM((c * 16,), jnp.float32),
            pltpu.VMEM((c * 16,), jnp.float32),
            pltpu.SemaphoreType.DMA,
        ],
        compiler_params=pltpu.CompilerParams(use_tc_tiling_on_sc=False),
    )


def _make_m0(VB):
    nrows = _NP0 * 32 // 128
    nv4 = _N0 // 4
    grid = -(-nv4 // VB)

    def body(*refs):
        g_refs = refs[:_L]
        w_ref, b_ref, out_ref = refs[_L], refs[_L + 1], refs[_L + 2]
        vals = [r[0] for r in g_refs]
        res = []
        for dv in range(4):
            cat = jnp.concatenate(
                [v[:, dv * 32:(dv + 1) * 32] for v in vals], axis=1)
            z = jnp.dot(cat, w_ref[...], preferred_element_type=jnp.float32)
            res.append(_elu(z + b_ref[...]))
        rows = [res[r // 2][:, None, (r % 2) * 128:(r % 2 + 1) * 128]
                for r in range(8)]
        out = jnp.concatenate(rows, axis=1)
        out_ref[...] = out.reshape(VB * 8, 128)

    in_specs = [
        pl.BlockSpec((1, VB, 128), (lambda i, l=l: (l, i, 0)))
        for l in range(_L)
    ]
    in_specs.append(pl.BlockSpec((_L * 32, 256), lambda i: (0, 0)))
    in_specs.append(pl.BlockSpec((1, 256), lambda i: (0, 0)))

    return pl.pallas_call(
        body,
        grid=(grid,),
        in_specs=in_specs,
        out_specs=pl.BlockSpec((VB * 8, 128), lambda i: (i, 0)),
        out_shape=jax.ShapeDtypeStruct((nv4 * 8, 128), jnp.float32),
        compiler_params=pltpu.CompilerParams(
            dimension_semantics=("arbitrary",)
        ),
    )


def _make_mm(N, Np, Cin, Cout, VBv):
    rv_in = 8 * Cin // 128
    per_row = 128 // Cin
    zc = per_row * Cout
    spr = zc // 128
    rv_out = rv_in * spr
    grid = -(-N // VBv)

    def body(*refs):
        g_refs = refs[:_L]
        w_ref, b_ref, out_ref = refs[_L], refs[_L + 1], refs[_L + 2]
        vals = [r[0].reshape(VBv, rv_in, 128) for r in g_refs]
        rows = []
        for h in range(rv_in):
            cat = jnp.concatenate([v[:, h, :] for v in vals], axis=1)
            z = jnp.dot(cat, w_ref[...], preferred_element_type=jnp.float32)
            z = _elu(z + b_ref[...])
            for s in range(spr):
                rows.append(z[:, None, s * 128:(s + 1) * 128])
        out = jnp.concatenate(rows, axis=1)
        out_ref[...] = out.reshape(VBv * rv_out, 128)

    in_specs = [
        pl.BlockSpec((1, VBv * rv_in, 128), (lambda i, l=l: (l, i, 0)))
        for l in range(_L)
    ]
    in_specs.append(pl.BlockSpec((_L * 128, zc), lambda i: (0, 0)))
    in_specs.append(pl.BlockSpec((1, zc), lambda i: (0, 0)))

    return pl.pallas_call(
        body,
        grid=(grid,),
        in_specs=in_specs,
        out_specs=pl.BlockSpec((VBv * rv_out, 128), lambda i: (i, 0)),
        out_shape=jax.ShapeDtypeStruct((N * rv_out, 128), jnp.float32),
        compiler_params=pltpu.CompilerParams(
            dimension_semantics=("arbitrary",)
        ),
    )


def _make_final(Mp):
    steps = 14
    vb = _N3 // steps
    kb = vb * 128

    def body(h_ref, w3_ref, b3_ref, w4_ref, b4_ref, xe_ref, y_ref, acc_ref):
        i = pl.program_id(0)

        @pl.when(i == 0)
        def _init():
            acc_ref[...] = jnp.zeros((_B, 256), jnp.float32)

        def vstep(v, acc):
            hv = h_ref[pl.ds(v * _B, _B), :]
            wv = w3_ref[pl.ds(v * 128, 128), :]
            return acc + jnp.dot(hv, wv, preferred_element_type=jnp.float32)

        acc_ref[...] = lax.fori_loop(0, vb, vstep, acc_ref[...])

        @pl.when(i == steps - 1)
        def _fin():
            xe = _elu(acc_ref[...] + b3_ref[...])
            xe_ref[...] = xe
            y_ref[...] = (
                jnp.dot(xe, w4_ref[...], preferred_element_type=jnp.float32)
                + b4_ref[...]
            )

    return pl.pallas_call(
        body,
        grid=(steps,),
        in_specs=[
            pl.BlockSpec((vb * _B, 128), lambda i: (i, 0)),
            pl.BlockSpec((kb, 256), lambda i: (i, 0)),
            pl.BlockSpec((1, 256), lambda i: (0, 0)),
            pl.BlockSpec((256, 64), lambda i: (0, 0)),
            pl.BlockSpec((1, 64), lambda i: (0, 0)),
        ],
        out_specs=[
            pl.BlockSpec((_B, 256), lambda i: (0, 0)),
            pl.BlockSpec((_B, 64), lambda i: (0, 0)),
        ],
        out_shape=[
            jax.ShapeDtypeStruct((_B, 256), jnp.float32),
            jax.ShapeDtypeStruct((_B, 64), jnp.float32),
        ],
        scratch_shapes=[pltpu.VMEM((_B, 256), jnp.float32)],
        compiler_params=pltpu.CompilerParams(
            dimension_semantics=("arbitrary",)
        ),
    )


def _pad_idx(idx, Np):
    t = idx.T.astype(jnp.int32)
    return jnp.pad(t, ((0, 0), (0, Np - t.shape[1])))


def _chunk_idx(idx, Np, c, ncs_pad):
    t = _pad_idx(idx, Np).reshape(-1, c)
    return jnp.pad(t, ((0, ncs_pad - t.shape[0]), (0, 0)))


def _pad_w(w, Np):
    t = w.T.astype(jnp.float32)
    t = jnp.pad(t, ((0, 0), (0, Np - t.shape[1])))
    return jnp.broadcast_to(t[:, :, None], (t.shape[0], Np, 16)).reshape(-1)


_NP0 = _rup(_N0, 128)
_MP0 = _rup(_N1, 64)
_NP1 = _rup(_N1, 128)
_MP1 = _rup(_N2, 64)
_NP2 = _rup(_N2, 128)
_MP2 = _rup(_N3, 32)


def kernel(x, spiral0, spiral1, spiral2, down_idx0, down_w0, down_idx1,
           down_w1, down_idx2, down_w2, W0, b0, W1, b1, W2, b2, W3, b3,
           W4, b4):
    xt = jnp.pad(x, ((0, 0), (0, 0), (0, 1)))
    xt = xt.transpose(1, 0, 2).reshape(_N0, _B * 4)

    gk0, ncs0 = _make_gather(_N0, _B * 4, _L, _NP0, 128)
    gk1, ncs1 = _make_gather(_MP0, _B * 32, _L, _NP1, 128)
    gk2, ncs2 = _make_gather(_MP1, _B * 64, _L, _NP2, 64)
    sp0 = _chunk_idx(spiral0, _NP0, 128, ncs0)
    sp1 = _chunk_idx(spiral1, _NP1, 128, ncs1)
    sp2 = _chunk_idx(spiral2, _NP2, 64, ncs2)
    d0 = _pad_idx(down_idx0, _MP0)
    d1 = _pad_idx(down_idx1, _MP1)
    d2 = _pad_idx(down_idx2, _MP2)
    w0p = _pad_w(down_w0, _MP0)
    w1p = _pad_w(down_w1, _MP1)
    w2p = _pad_w(down_w2, _MP2)

    W0p = jnp.pad(W0.reshape(_L, 3, 32), ((0, 0), (0, 1), (0, 0)))
    eyeB = jnp.eye(_B, dtype=jnp.float32)
    W0k = jax.vmap(lambda w: jnp.kron(eyeB, w))(W0p)
    W0k = W0k.reshape(_L * _B * 4, _B * 32)
    b0k = jnp.tile(b0, _B).reshape(1, _B * 32)
    eye4 = jnp.eye(4, dtype=jnp.float32)
    eye2 = jnp.eye(2, dtype=jnp.float32)
    Wk1 = jax.vmap(lambda w: jnp.kron(eye4, w))(W1.reshape(_L, 32, 64))
    Wk1 = Wk1.reshape(_L * 128, 256)
    bk1 = jnp.tile(b1, 4).reshape(1, 256)
    Wk2 = jax.vmap(lambda w: jnp.kron(eye2, w))(W2.reshape(_L, 64, 128))
    Wk2 = Wk2.reshape(_L * 128, 256)
    bk2 = jnp.tile(b2, 2).reshape(1, 256)
    b3r = b3.reshape(1, 256)
    b4r = b4.reshape(1, 64)

    g0 = gk0(xt, sp0)
    h0 = _make_m0(512)(*([g0.reshape(_L, _NP0 * 32 // 128, 128)] * _L),
                       W0k, b0k)
    h0p = _make_pool(_N0, _B * 32, _MP0, 64)(
        h0.reshape(_N0, _B * 32), d0, w0p)

    g1 = gk1(h0p, sp1)
    h1 = _make_mm(_N1, _NP1, 32, 64, 512)(
        *([g1.reshape(_L, _NP1 * 2, 128)] * _L), Wk1, bk1)
    h1p = _make_pool(_N1, _B * 64, _MP1, 64)(
        h1.reshape(_N1, _B * 64), d1, w1p)

    g2 = gk2(h1p, sp2)
    h2 = _make_mm(_N2, _NP2, 64, 128, 256)(
        *([g2.reshape(_L, _NP2 * 4, 128)] * _L), Wk2, bk2)
    h2p = _make_pool(_N2, _B * 128, _MP2, 32)(
        h2.reshape(_N2, _B * 128), d2, w2p)

    xe, y = _make_final(_MP2)(h2p.reshape(_MP2 * _B, 128), W3, b3r, W4, b4r)
    return (xe, y)


# speedup vs baseline: 9.0729x; 1.1306x over previous
import jax
import jax.numpy as jnp
from jax import lax
from jax.experimental import pallas as pl
from jax.experimental.pallas import tpu as pltpu
from jax.experimental.pallas import tpu_sc as plsc

NC, NS = 2, 16
NW = NC * NS

_B = 8
_N0, _N1, _N2, _N3 = 50000, 12500, 3125, 784
_L = 9
_K = 3


def _rup(n, m):
    return ((n + m - 1) // m) * m


def _elu(z):
    return jnp.where(z > 0, z, jnp.exp(jnp.minimum(z, 0.0)) - 1.0)


def _sc_mesh():
    return plsc.VectorSubcoreMesh(
        core_axis_name="c", subcore_axis_name="s", num_cores=NC, num_subcores=NS
    )


def _make_gather(N, D, L, Np, c):
    total = L * Np // c
    nj = -(-total // NW)
    nj += nj % 2
    ncs_pad = nj * NW

    def body(table, idx, out, ib, b0, b1, gs0, gs1, ss0, ss1):
        wid = lax.axis_index("s") * NC + lax.axis_index("c")
        base = wid * nj
        bufs = (b0, b1)
        gsem = (gs0, gs1)
        ssem = (ss0, ss1)
        pltpu.sync_copy(idx.at[pl.ds(base, nj), :], ib)

        def gather(t, b):
            pltpu.async_copy(table.at[ib.at[t]], bufs[b], gsem[b])

        def store(t, b):
            pltpu.async_copy(
                bufs[b], out.at[pl.ds((base + t) * c, c), :], ssem[b])

        @pl.when(base < total)
        def _p0():
            gather(0, 0)

        @pl.when(base + 1 < total)
        def _p1():
            gather(1, 1)

        def grp(g, carry):
            for b in range(2):
                t = g * 2 + b

                @pl.when(base + t < total)
                def _fin(t=t, b=b):
                    pltpu.make_async_copy(
                        table.at[ib.at[t]], bufs[b], gsem[b]).wait()
                    store(t, b)

                @pl.when((base + t + 2 < total) & (t + 2 < nj))
                def _nxt(t=t, b=b):
                    pltpu.make_async_copy(
                        bufs[b], out.at[pl.ds(0, c), :], ssem[b]).wait()
                    gather(t + 2, b)

            return carry

        lax.fori_loop(0, nj // 2, grp, 0)

        for b in range(2):
            @pl.when(base + b < total)
            def _dr(b=b):
                pltpu.make_async_copy(
                    bufs[b], out.at[pl.ds(0, c), :], ssem[b]).wait()

    return pl.kernel(
        body,
        out_type=jax.ShapeDtypeStruct((L * Np, D), jnp.float32),
        mesh=_sc_mesh(),
        scratch_types=[
            pltpu.VMEM((nj, c), jnp.int32),
            pltpu.VMEM((c, D), jnp.float32),
            pltpu.VMEM((c, D), jnp.float32),
            pltpu.SemaphoreType.DMA,
            pltpu.SemaphoreType.DMA,
            pltpu.SemaphoreType.DMA,
            pltpu.SemaphoreType.DMA,
        ],
        compiler_params=pltpu.CompilerParams(use_tc_tiling_on_sc=False),
    ), ncs_pad


def _make_pool(N, D, Mp, c):
    ncs = Mp // c
    nj = -(-ncs // NW)
    nj += nj % 2
    per = nj * c
    Mp_pad = NW * per

    def body(table, idx, wts, out,
             ib, wb, a00, a01, a02, a10, a11, a12, gs0, gs1, ss0, ss1):
        wid = lax.axis_index("s") * NC + lax.axis_index("c")
        base = wid * nj
        bufs = ((a00, a01, a02), (a10, a11, a12))
        gsem = (gs0, gs1)
        ssem = (ss0, ss1)
        for k in range(_K):
            pltpu.sync_copy(idx.at[k, pl.ds(base * c, per)], ib.at[k])
            pltpu.sync_copy(
                wts.at[pl.ds((k * Mp_pad + base * c) * 16, per * 16)],
                wb.at[k])

        def gathers(t, b):
            for k in range(_K):
                pltpu.async_copy(
                    table.at[ib.at[k, pl.ds(t * c, c)]], bufs[b][k], gsem[b])

        def wait_gathers(t, b):
            for k in range(_K):
                pltpu.make_async_copy(
                    table.at[ib.at[k, pl.ds(t * c, c)]], bufs[b][k],
                    gsem[b]).wait()

        def combine(t, b):
            a0, a1, a2 = bufs[b]

            def row(r, carry2):
                w0 = wb[0, pl.ds((t * c + r) * 16, 16)]
                w1 = wb[1, pl.ds((t * c + r) * 16, 16)]
                w2 = wb[2, pl.ds((t * c + r) * 16, 16)]
                for q in range(D // 16):
                    s = q * 16
                    v = (
                        a0[r, pl.ds(s, 16)] * w0
                        + a1[r, pl.ds(s, 16)] * w1
                        + a2[r, pl.ds(s, 16)] * w2
                    )
                    a0[r, pl.ds(s, 16)] = v
                return carry2

            lax.fori_loop(0, c, row, 0)

        @pl.when(base < ncs)
        def _p0():
            gathers(0, 0)

        @pl.when(base + 1 < ncs)
        def _p1():
            gathers(1, 1)

        def grp(g, carry):
            for b in range(2):
                t = g * 2 + b

                @pl.when(base + t < ncs)
                def _fin(t=t, b=b):
                    wait_gathers(t, b)
                    combine(t, b)
                    pltpu.async_copy(
                        bufs[b][0], out.at[pl.ds((base + t) * c, c), :],
                        ssem[b])

                @pl.when((base + t + 2 < ncs) & (t + 2 < nj))
                def _nxt(t=t, b=b):
                    pltpu.make_async_copy(
                        bufs[b][0], out.at[pl.ds(0, c), :], ssem[b]).wait()
                    gathers(t + 2, b)

            return carry

        lax.fori_loop(0, nj // 2, grp, 0)

        for b in range(2):
            @pl.when(base + b < ncs)
            def _dr(b=b):
                pltpu.make_async_copy(
                    bufs[b][0], out.at[pl.ds(0, c), :], ssem[b]).wait()

    return pl.kernel(
        body,
        out_type=jax.ShapeDtypeStruct((Mp, D), jnp.float32),
        mesh=_sc_mesh(),
        scratch_types=[
            pltpu.VMEM((_K, per), jnp.int32),
            pltpu.VMEM((_K, per * 16), jnp.float32),
            pltpu.VMEM((c, D), jnp.float32),
            pltpu.VMEM((c, D), jnp.float32),
            pltpu.VMEM((c, D), jnp.float32),
            pltpu.VMEM((c, D), jnp.float32),
            pltpu.VMEM((c, D), jnp.float32),
            pltpu.VMEM((c, D), jnp.float32),
            pltpu.SemaphoreType.DMA,
            pltpu.SemaphoreType.DMA,
            pltpu.SemaphoreType.DMA,
            pltpu.SemaphoreType.DMA,
        ],
        compiler_params=pltpu.CompilerParams(use_tc_tiling_on_sc=False),
    ), Mp_pad


def _make_m0(VB):
    nrows = _NP0 * 32 // 128
    nv4 = _N0 // 4
    grid = -(-nv4 // VB)

    def body(*refs):
        g_refs = refs[:_L]
        w_ref, b_ref, out_ref = refs[_L], refs[_L + 1], refs[_L + 2]
        vals = [r[0] for r in g_refs]
        res = []
        for dv in range(4):
            cat = jnp.concatenate(
                [v[:, dv * 32:(dv + 1) * 32] for v in vals], axis=1)
            z = jnp.dot(cat, w_ref[...], preferred_element_type=jnp.float32)
            res.append(_elu(z + b_ref[...]))
        rows = [res[r // 2][:, None, (r % 2) * 128:(r % 2 + 1) * 128]
                for r in range(8)]
        out = jnp.concatenate(rows, axis=1)
        out_ref[...] = out.reshape(VB * 8, 128)

    in_specs = [
        pl.BlockSpec((1, VB, 128), (lambda i, l=l: (l, i, 0)))
        for l in range(_L)
    ]
    in_specs.append(pl.BlockSpec((_L * 32, 256), lambda i: (0, 0)))
    in_specs.append(pl.BlockSpec((1, 256), lambda i: (0, 0)))

    return pl.pallas_call(
        body,
        grid=(grid,),
        in_specs=in_specs,
        out_specs=pl.BlockSpec((VB * 8, 128), lambda i: (i, 0)),
        out_shape=jax.ShapeDtypeStruct((nv4 * 8, 128), jnp.float32),
        compiler_params=pltpu.CompilerParams(
            dimension_semantics=("arbitrary",)
        ),
    )


def _make_mm(N, Np, Cin, Cout, VBv):
    rv_in = 8 * Cin // 128
    per_row = 128 // Cin
    zc = per_row * Cout
    spr = zc // 128
    rv_out = rv_in * spr
    grid = -(-N // VBv)

    def body(*refs):
        g_refs = refs[:_L]
        w_ref, b_ref, out_ref = refs[_L], refs[_L + 1], refs[_L + 2]
        vals = [r[0].reshape(VBv, rv_in, 128) for r in g_refs]
        rows = []
        for h in range(rv_in):
            cat = jnp.concatenate([v[:, h, :] for v in vals], axis=1)
            z = jnp.dot(cat, w_ref[...], preferred_element_type=jnp.float32)
            z = _elu(z + b_ref[...])
            for s in range(spr):
                rows.append(z[:, None, s * 128:(s + 1) * 128])
        out = jnp.concatenate(rows, axis=1)
        out_ref[...] = out.reshape(VBv * rv_out, 128)

    in_specs = [
        pl.BlockSpec((1, VBv * rv_in, 128), (lambda i, l=l: (l, i, 0)))
        for l in range(_L)
    ]
    in_specs.append(pl.BlockSpec((_L * 128, zc), lambda i: (0, 0)))
    in_specs.append(pl.BlockSpec((1, zc), lambda i: (0, 0)))

    return pl.pallas_call(
        body,
        grid=(grid,),
        in_specs=in_specs,
        out_specs=pl.BlockSpec((VBv * rv_out, 128), lambda i: (i, 0)),
        out_shape=jax.ShapeDtypeStruct((N * rv_out, 128), jnp.float32),
        compiler_params=pltpu.CompilerParams(
            dimension_semantics=("arbitrary",)
        ),
    )


def _make_final(Mp):
    steps = 14
    vb = _N3 // steps
    kb = vb * 128

    def body(h_ref, w3_ref, b3_ref, w4_ref, b4_ref, xe_ref, y_ref, acc_ref):
        i = pl.program_id(0)

        @pl.when(i == 0)
        def _init():
            acc_ref[...] = jnp.zeros((_B, 256), jnp.float32)

        def vstep(v, acc):
            hv = h_ref[pl.ds(v * _B, _B), :]
            wv = w3_ref[pl.ds(v * 128, 128), :]
            return acc + jnp.dot(hv, wv, preferred_element_type=jnp.float32)

        acc_ref[...] = lax.fori_loop(0, vb, vstep, acc_ref[...])

        @pl.when(i == steps - 1)
        def _fin():
            xe = _elu(acc_ref[...] + b3_ref[...])
            xe_ref[...] = xe
            y_ref[...] = (
                jnp.dot(xe, w4_ref[...], preferred_element_type=jnp.float32)
                + b4_ref[...]
            )

    return pl.pallas_call(
        body,
        grid=(steps,),
        in_specs=[
            pl.BlockSpec((vb * _B, 128), lambda i: (i, 0)),
            pl.BlockSpec((kb, 256), lambda i: (i, 0)),
            pl.BlockSpec((1, 256), lambda i: (0, 0)),
            pl.BlockSpec((256, 64), lambda i: (0, 0)),
            pl.BlockSpec((1, 64), lambda i: (0, 0)),
        ],
        out_specs=[
            pl.BlockSpec((_B, 256), lambda i: (0, 0)),
            pl.BlockSpec((_B, 64), lambda i: (0, 0)),
        ],
        out_shape=[
            jax.ShapeDtypeStruct((_B, 256), jnp.float32),
            jax.ShapeDtypeStruct((_B, 64), jnp.float32),
        ],
        scratch_shapes=[pltpu.VMEM((_B, 256), jnp.float32)],
        compiler_params=pltpu.CompilerParams(
            dimension_semantics=("arbitrary",)
        ),
    )


def _pad_idx(idx, Np):
    t = idx.T.astype(jnp.int32)
    return jnp.pad(t, ((0, 0), (0, Np - t.shape[1])))


def _chunk_idx(idx, Np, c, ncs_pad):
    t = _pad_idx(idx, Np).reshape(-1, c)
    return jnp.pad(t, ((0, ncs_pad - t.shape[0]), (0, 0)))


def _pad_w(w, Np):
    t = w.T.astype(jnp.float32)
    t = jnp.pad(t, ((0, 0), (0, Np - t.shape[1])))
    return jnp.broadcast_to(t[:, :, None], (t.shape[0], Np, 16)).reshape(-1)


_NP0 = _rup(_N0, 128)
_MP0 = _rup(_N1, 64)
_NP1 = _rup(_N1, 128)
_MP1 = _rup(_N2, 64)
_NP2 = _rup(_N2, 128)
_MP2 = _rup(_N3, 32)


def kernel(x, spiral0, spiral1, spiral2, down_idx0, down_w0, down_idx1,
           down_w1, down_idx2, down_w2, W0, b0, W1, b1, W2, b2, W3, b3,
           W4, b4):
    xt = jnp.pad(x, ((0, 0), (0, 0), (0, 1)))
    xt = xt.transpose(1, 0, 2).reshape(_N0, _B * 4)

    gk0, ncs0 = _make_gather(_N0, _B * 4, _L, _NP0, 128)
    gk1, ncs1 = _make_gather(_MP0, _B * 32, _L, _NP1, 128)
    gk2, ncs2 = _make_gather(_MP1, _B * 64, _L, _NP2, 64)
    sp0 = _chunk_idx(spiral0, _NP0, 128, ncs0)
    sp1 = _chunk_idx(spiral1, _NP1, 128, ncs1)
    sp2 = _chunk_idx(spiral2, _NP2, 64, ncs2)
    pk0, mpad0 = _make_pool(_N0, _B * 32, _MP0, 64)
    pk1, mpad1 = _make_pool(_N1, _B * 64, _MP1, 32)
    pk2, mpad2 = _make_pool(_N2, _B * 128, _MP2, 16)
    d0 = _pad_idx(down_idx0, mpad0)
    d1 = _pad_idx(down_idx1, mpad1)
    d2 = _pad_idx(down_idx2, mpad2)
    w0p = _pad_w(down_w0, mpad0)
    w1p = _pad_w(down_w1, mpad1)
    w2p = _pad_w(down_w2, mpad2)

    W0p = jnp.pad(W0.reshape(_L, 3, 32), ((0, 0), (0, 1), (0, 0)))
    eyeB = jnp.eye(_B, dtype=jnp.float32)
    W0k = jax.vmap(lambda w: jnp.kron(eyeB, w))(W0p)
    W0k = W0k.reshape(_L * _B * 4, _B * 32)
    b0k = jnp.tile(b0, _B).reshape(1, _B * 32)
    eye4 = jnp.eye(4, dtype=jnp.float32)
    eye2 = jnp.eye(2, dtype=jnp.float32)
    Wk1 = jax.vmap(lambda w: jnp.kron(eye4, w))(W1.reshape(_L, 32, 64))
    Wk1 = Wk1.reshape(_L * 128, 256)
    bk1 = jnp.tile(b1, 4).reshape(1, 256)
    Wk2 = jax.vmap(lambda w: jnp.kron(eye2, w))(W2.reshape(_L, 64, 128))
    Wk2 = Wk2.reshape(_L * 128, 256)
    bk2 = jnp.tile(b2, 2).reshape(1, 256)
    b3r = b3.reshape(1, 256)
    b4r = b4.reshape(1, 64)

    g0 = gk0(xt, sp0)
    h0 = _make_m0(512)(*([g0.reshape(_L, _NP0 * 32 // 128, 128)] * _L),
                       W0k, b0k)
    h0p = pk0(h0.reshape(_N0, _B * 32), d0, w0p)

    g1 = gk1(h0p, sp1)
    h1 = _make_mm(_N1, _NP1, 32, 64, 512)(
        *([g1.reshape(_L, _NP1 * 2, 128)] * _L), Wk1, bk1)
    h1p = pk1(h1.reshape(_N1, _B * 64), d1, w1p)

    g2 = gk2(h1p, sp2)
    h2 = _make_mm(_N2, _NP2, 64, 128, 256)(
        *([g2.reshape(_L, _NP2 * 4, 128)] * _L), Wk2, bk2)
    h2p = pk2(h2.reshape(_N2, _B * 128), d2, w2p)

    xe, y = _make_final(_MP2)(h2p.reshape(_MP2 * _B, 128), W3, b3r, W4, b4r)
    return (xe, y)
